# baseline (speedup 1.0000x reference)
import functools

import jax
import jax.numpy as jnp
from jax import lax
from jax.experimental import pallas as pl
from jax.experimental.pallas import tpu as pltpu
from jax.experimental.pallas import tpu_sc as plsc

N_NODES = 10000
N_EDGES = 160000
D = 256
DH = 128
N_GRAPHS = 64
N_CLASSES = 40
DEPTH = 3
EPS = 1e-5

ROWS_BLK = 2000
N_BLKS = N_NODES // ROWS_BLK

NS = 16
EDGES_PER_SUB = N_EDGES // NS
CHUNK = 80
N_CHUNKS = EDGES_PER_SUB // CHUNK
ROWS_PER_SUB = 624
ROWS_TAIL = N_NODES - NS * ROWS_PER_SUB


def _enc_compute(h, wrel_ref, wroot_ref, b_ref,
                 t0_ref, t1_ref, r0_ref, r1_ref, apply_relu):
    if apply_relu:
        h = jnp.maximum(h, 0.0)
    t = jnp.dot(h, wrel_ref[...], preferred_element_type=jnp.float32)
    r = jnp.dot(h, wroot_ref[...], preferred_element_type=jnp.float32) + b_ref[...]
    t0_ref[...] = t[:, :DH]
    t1_ref[...] = t[:, DH:]
    r0_ref[...] = r[:, :DH]
    r1_ref[...] = r[:, DH:]


_half_out = [jax.ShapeDtypeStruct((N_NODES, DH), jnp.float32)] * 4
_half_spec = pl.BlockSpec((ROWS_BLK, DH), lambda i: (i, 0))
_w_spec = pl.BlockSpec((D, D), lambda i: (0, 0))
_b_spec = pl.BlockSpec((1, D), lambda i: (0, 0))


def _enc_matmul_single(h, wrel, wroot, b):
    def body(h_ref, wrel_ref, wroot_ref, b_ref, *outs):
        _enc_compute(h_ref[...], wrel_ref, wroot_ref, b_ref, *outs,
                     apply_relu=False)
    return pl.pallas_call(
        body,
        grid=(N_BLKS,),
        in_specs=[pl.BlockSpec((ROWS_BLK, D), lambda i: (i, 0)),
                  _w_spec, _w_spec, _b_spec],
        out_specs=[_half_spec] * 4,
        out_shape=_half_out,
    )(h, wrel, wroot, b)


def _enc_matmul_halves(h0, h1, wrel, wroot, b):
    def body(h0_ref, h1_ref, wrel_ref, wroot_ref, b_ref, *outs):
        h = jnp.concatenate([h0_ref[...], h1_ref[...]], axis=1)
        _enc_compute(h, wrel_ref, wroot_ref, b_ref, *outs, apply_relu=True)
    return pl.pallas_call(
        body,
        grid=(N_BLKS,),
        in_specs=[_half_spec, _half_spec, _w_spec, _w_spec, _b_spec],
        out_specs=[_half_spec] * 4,
        out_shape=_half_out,
    )(h0, h1, wrel, wroot, b)


_sc_mesh = plsc.VectorSubcoreMesh(core_axis_name="c", subcore_axis_name="s")


@functools.partial(
    pl.kernel,
    mesh=_sc_mesh,
    out_type=[jax.ShapeDtypeStruct((N_NODES, DH), jnp.float32),
              jax.ShapeDtypeStruct((N_NODES, DH), jnp.float32)],
    scratch_types=[
        pltpu.VMEM((EDGES_PER_SUB,), jnp.int32),
        pltpu.VMEM((CHUNK,), jnp.int32),
        pltpu.VMEM((CHUNK,), jnp.int32),
        pltpu.VMEM((CHUNK,), jnp.int32),
        pltpu.VMEM((CHUNK, DH), jnp.float32),
        pltpu.VMEM((CHUNK, DH), jnp.float32),
        pltpu.VMEM((CHUNK, DH), jnp.float32),
        pltpu.VMEM_SHARED((N_NODES, DH), jnp.float32),
        pltpu.SemaphoreType.DMA,
        pltpu.SemaphoreType.DMA,
        pltpu.SemaphoreType.DMA,
        pltpu.SemaphoreType.DMA,
        pltpu.SemaphoreType.DMA,
        pltpu.SemaphoreType.DMA,
        pltpu.SemaphoreType.DMA,
        pltpu.SemaphoreType.DMA,
        pltpu.SemaphoreType.DMA,
    ],
)
def _sc_aggregate(t0_hbm, t1_hbm, r0_hbm, r1_hbm, src_hbm, dst_hbm,
                  out0_hbm, out1_hbm, idx_src, d0, d1, d2, rb0, rb1, rb2,
                  shared, sg0, sg1, sg2, si0, si1, si2, sh0, sh1, sh2):
    c = lax.axis_index("c")
    s = lax.axis_index("s")
    row0 = s * ROWS_PER_SUB

    tail0 = NS * ROWS_PER_SUB

    @pl.when(c == 0)
    def _():
        pltpu.async_copy(r0_hbm.at[pl.ds(row0, ROWS_PER_SUB)],
                         shared.at[pl.ds(row0, ROWS_PER_SUB)], sg0)

        @pl.when(s == NS - 1)
        def _():
            pltpu.async_copy(r0_hbm.at[pl.ds(tail0, ROWS_TAIL)],
                             shared.at[pl.ds(tail0, ROWS_TAIL)], sg1)

    @pl.when(c == 1)
    def _():
        pltpu.async_copy(r1_hbm.at[pl.ds(row0, ROWS_PER_SUB)],
                         shared.at[pl.ds(row0, ROWS_PER_SUB)], sg0)

        @pl.when(s == NS - 1)
        def _():
            pltpu.async_copy(r1_hbm.at[pl.ds(tail0, ROWS_TAIL)],
                             shared.at[pl.ds(tail0, ROWS_TAIL)], sg1)

    pltpu.sync_copy(src_hbm.at[s], idx_src)
    pltpu.make_async_copy(r0_hbm.at[pl.ds(row0, ROWS_PER_SUB)],
                          shared.at[pl.ds(row0, ROWS_PER_SUB)], sg0).wait()

    @pl.when(s == NS - 1)
    def _():
        pltpu.make_async_copy(r0_hbm.at[pl.ds(tail0, ROWS_TAIL)],
                              shared.at[pl.ds(tail0, ROWS_TAIL)], sg1).wait()

    plsc.subcore_barrier()

    dbufs = (d0, d1, d2)
    rbufs = (rb0, rb1, rb2)
    gsems = (sg0, sg1, sg2)
    isems = (si0, si1, si2)

    H = CHUNK // 2
    hsems = (sh0, sh1, sh2)

    def start_gather(ch, buf, sem, sem2):
        sla = idx_src.at[pl.ds(ch * CHUNK, H)]
        slb = idx_src.at[pl.ds(ch * CHUNK + H, H)]

        @pl.when(c == 0)
        def _():
            pltpu.async_copy(t0_hbm.at[sla], buf.at[pl.ds(0, H)], sem)
            pltpu.async_copy(t0_hbm.at[slb], buf.at[pl.ds(H, H)], sem2)

        @pl.when(c == 1)
        def _():
            pltpu.async_copy(t1_hbm.at[sla], buf.at[pl.ds(0, H)], sem)
            pltpu.async_copy(t1_hbm.at[slb], buf.at[pl.ds(H, H)], sem2)

    def wait_gather(ch, buf, sem, sem2):
        sla = idx_src.at[pl.ds(ch * CHUNK, H)]
        slb = idx_src.at[pl.ds(ch * CHUNK + H, H)]
        pltpu.make_async_copy(t0_hbm.at[sla], buf.at[pl.ds(0, H)], sem).wait()
        pltpu.make_async_copy(t0_hbm.at[slb], buf.at[pl.ds(H, H)], sem2).wait()

    def start_didx(ch, buf, sem):
        sl = dst_hbm.at[pl.ds(s * EDGES_PER_SUB + ch * CHUNK, CHUNK)]
        pltpu.async_copy(sl, buf, sem)

    def wait_didx(ch, buf, sem):
        sl = dst_hbm.at[pl.ds(s * EDGES_PER_SUB + ch * CHUNK, CHUNK)]
        pltpu.make_async_copy(sl, buf, sem).wait()

    for p in range(3):
        start_didx(p, dbufs[p], isems[p])
        start_gather(p, rbufs[p], gsems[p], hsems[p])

    def step(ch, b, prefetch):
        wait_gather(ch, rbufs[b], gsems[b], hsems[b])
        wait_didx(ch, dbufs[b], isems[b])
        pltpu.sync_copy(rbufs[b], shared.at[dbufs[b]], add=True)
        if prefetch:
            @pl.when(ch + 3 < N_CHUNKS)
            def _():
                start_didx(ch + 3, dbufs[b], isems[b])
                start_gather(ch + 3, rbufs[b], gsems[b], hsems[b])

    def body(g, carry):
        ch0 = g * 3
        step(ch0, 0, True)
        step(ch0 + 1, 1, True)
        step(ch0 + 2, 2, True)
        return carry

    lax.fori_loop(0, N_CHUNKS // 3, body, 0)

    step(N_CHUNKS - 2, (N_CHUNKS - 2) % 3, False)
    step(N_CHUNKS - 1, (N_CHUNKS - 1) % 3, False)

    plsc.subcore_barrier()

    @pl.when(c == 0)
    def _():
        pltpu.sync_copy(shared.at[pl.ds(row0, ROWS_PER_SUB)],
                        out0_hbm.at[pl.ds(row0, ROWS_PER_SUB)])

        @pl.when(s == NS - 1)
        def _():
            pltpu.sync_copy(shared.at[pl.ds(tail0, ROWS_TAIL)],
                            out0_hbm.at[pl.ds(tail0, ROWS_TAIL)])

    @pl.when(c == 1)
    def _():
        pltpu.sync_copy(shared.at[pl.ds(row0, ROWS_PER_SUB)],
                        out1_hbm.at[pl.ds(row0, ROWS_PER_SUB)])

        @pl.when(s == NS - 1)
        def _():
            pltpu.sync_copy(shared.at[pl.ds(tail0, ROWS_TAIL)],
                            out1_hbm.at[pl.ds(tail0, ROWS_TAIL)])


def _pool_mlp_body(a0_ref, a1_ref, batch_ref,
                   w1_ref, b1_ref, g1_ref, be1_ref,
                   w2_ref, b2_ref, g2_ref, be2_ref,
                   out_ref, pooled_acc, counts_acc):
    i = pl.program_id(0)

    @pl.when(i == 0)
    def _():
        pooled_acc[...] = jnp.zeros_like(pooled_acc)
        counts_acc[...] = jnp.zeros_like(counts_acc)

    h = jnp.maximum(
        jnp.concatenate([a0_ref[...], a1_ref[...]], axis=1), 0.0)
    b_ids = batch_ref[0, 0, :]
    gids = lax.broadcasted_iota(jnp.int32, (ROWS_BLK, N_GRAPHS), 1)
    m = (b_ids[:, None] == gids).astype(jnp.float32)
    pooled_acc[...] += lax.dot_general(
        m, h, (((0,), (0,)), ((), ())), precision=lax.Precision.HIGHEST,
        preferred_element_type=jnp.float32)
    counts_acc[...] += lax.dot_general(
        m, jnp.ones((ROWS_BLK, D), jnp.float32),
        (((0,), (0,)), ((), ())), precision=lax.Precision.HIGHEST,
        preferred_element_type=jnp.float32)

    @pl.when(i == N_BLKS - 1)
    def _():
        pooled = pooled_acc[...] / jnp.maximum(counts_acc[...], 1.0)
        h1 = jnp.maximum(
            jnp.dot(pooled, w1_ref[...], preferred_element_type=jnp.float32)
            + b1_ref[...], 0.0)
        mu1 = jnp.mean(h1, axis=0, keepdims=True)
        var1 = jnp.mean((h1 - mu1) ** 2, axis=0, keepdims=True)
        h1n = (h1 - mu1) / jnp.sqrt(var1 + EPS) * g1_ref[...] + be1_ref[...]
        h2 = jnp.maximum(
            jnp.dot(h1n, w2_ref[...], preferred_element_type=jnp.float32)
            + b2_ref[...], 0.0)
        mu2 = jnp.mean(h2, axis=0, keepdims=True)
        var2 = jnp.mean((h2 - mu2) ** 2, axis=0, keepdims=True)
        h2n = (h2 - mu2) / jnp.sqrt(var2 + EPS) * g2_ref[...] + be2_ref[...]
        mx = jnp.max(h2n, axis=1, keepdims=True)
        lse = jnp.log(jnp.sum(jnp.exp(h2n - mx), axis=1, keepdims=True))
        out_ref[...] = h2n - mx - lse


def _pool_mlp(a0, a1, batch3, w1, b1, g1, be1, w2, b2, g2, be2):
    vec_d = pl.BlockSpec((1, D), lambda i: (0, 0))
    vec_c = pl.BlockSpec((1, N_CLASSES), lambda i: (0, 0))
    return pl.pallas_call(
        _pool_mlp_body,
        grid=(N_BLKS,),
        in_specs=[
            _half_spec, _half_spec,
            pl.BlockSpec((1, 1, ROWS_BLK), lambda i: (i, 0, 0)),
            pl.BlockSpec((D, D), lambda i: (0, 0)), vec_d, vec_d, vec_d,
            pl.BlockSpec((D, N_CLASSES), lambda i: (0, 0)), vec_c, vec_c, vec_c,
        ],
        out_specs=pl.BlockSpec((N_GRAPHS, N_CLASSES), lambda i: (0, 0)),
        out_shape=jax.ShapeDtypeStruct((N_GRAPHS, N_CLASSES), jnp.float32),
        scratch_shapes=[pltpu.VMEM((N_GRAPHS, D), jnp.float32),
                        pltpu.VMEM((N_GRAPHS, D), jnp.float32)],
    )(a0, a1, batch3, w1, b1, g1, be1, w2, b2, g2, be2)


def kernel(x, edge_index, batch, enc_W_root, enc_W_rel, enc_b,
           mlp_W1, mlp_b1, mlp_g1, mlp_be1,
           mlp_W2, mlp_b2, mlp_g2, mlp_be2):
    src = edge_index[0].reshape(NS, EDGES_PER_SUB)
    dst = edge_index[1]
    batch3 = batch.reshape(N_BLKS, 1, ROWS_BLK)

    h0 = h1 = None
    for i in range(DEPTH):
        wrel = enc_W_rel[i]
        wroot = enc_W_root[i]
        b = enc_b[i].reshape(1, D)
        if i == 0:
            t0, t1, r0, r1 = _enc_matmul_single(x, wrel, wroot, b)
        else:
            t0, t1, r0, r1 = _enc_matmul_halves(h0, h1, wrel, wroot, b)
        h0, h1 = _sc_aggregate(t0, t1, r0, r1, src, dst)

    return _pool_mlp(
        h0, h1, batch3,
        mlp_W1, mlp_b1.reshape(1, D),
        mlp_g1.reshape(1, D), mlp_be1.reshape(1, D),
        mlp_W2, mlp_b2.reshape(1, N_CLASSES), mlp_g2.reshape(1, N_CLASSES),
        mlp_be2.reshape(1, N_CLASSES))


# speedup vs baseline: 1.0086x; 1.0086x over previous
import functools

import jax
import jax.numpy as jnp
from jax import lax
from jax.experimental import pallas as pl
from jax.experimental.pallas import tpu as pltpu
from jax.experimental.pallas import tpu_sc as plsc

N_NODES = 10000
N_EDGES = 160000
D = 256
DH = 128
N_GRAPHS = 64
N_CLASSES = 40
DEPTH = 3
EPS = 1e-5

ROWS_BLK = 2000
N_BLKS = N_NODES // ROWS_BLK

NS = 16
EDGES_PER_SUB = N_EDGES // NS
CHUNK = 80
N_CHUNKS = EDGES_PER_SUB // CHUNK
ROWS_PER_SUB = 624
ROWS_TAIL = N_NODES - NS * ROWS_PER_SUB


def _enc_compute(h, wrel_ref, wroot_ref, b_ref,
                 t0_ref, t1_ref, r0_ref, r1_ref, apply_relu):
    if apply_relu:
        h = jnp.maximum(h, 0.0)
    t = jnp.dot(h, wrel_ref[...], preferred_element_type=jnp.float32)
    r = jnp.dot(h, wroot_ref[...], preferred_element_type=jnp.float32) + b_ref[...]
    t0_ref[...] = t[:, :DH]
    t1_ref[...] = t[:, DH:]
    r0_ref[...] = r[:, :DH]
    r1_ref[...] = r[:, DH:]


_half_out = [jax.ShapeDtypeStruct((N_NODES, DH), jnp.float32)] * 4
_half_spec = pl.BlockSpec((ROWS_BLK, DH), lambda i: (i, 0))
_w_spec = pl.BlockSpec((D, D), lambda i: (0, 0))
_b_spec = pl.BlockSpec((1, D), lambda i: (0, 0))


def _enc_matmul_single(h, wrel, wroot, b):
    def body(h_ref, wrel_ref, wroot_ref, b_ref, *outs):
        _enc_compute(h_ref[...], wrel_ref, wroot_ref, b_ref, *outs,
                     apply_relu=False)
    return pl.pallas_call(
        body,
        grid=(N_BLKS,),
        in_specs=[pl.BlockSpec((ROWS_BLK, D), lambda i: (i, 0)),
                  _w_spec, _w_spec, _b_spec],
        out_specs=[_half_spec] * 4,
        out_shape=_half_out,
    )(h, wrel, wroot, b)


def _enc_matmul_halves(h0, h1, wrel, wroot, b):
    def body(h0_ref, h1_ref, wrel_ref, wroot_ref, b_ref, *outs):
        h = jnp.concatenate([h0_ref[...], h1_ref[...]], axis=1)
        _enc_compute(h, wrel_ref, wroot_ref, b_ref, *outs, apply_relu=True)
    return pl.pallas_call(
        body,
        grid=(N_BLKS,),
        in_specs=[_half_spec, _half_spec, _w_spec, _w_spec, _b_spec],
        out_specs=[_half_spec] * 4,
        out_shape=_half_out,
    )(h0, h1, wrel, wroot, b)


_sc_mesh = plsc.VectorSubcoreMesh(core_axis_name="c", subcore_axis_name="s")


@functools.partial(
    pl.kernel,
    mesh=_sc_mesh,
    out_type=[jax.ShapeDtypeStruct((N_NODES, DH), jnp.float32),
              jax.ShapeDtypeStruct((N_NODES, DH), jnp.float32)],
    scratch_types=[
        pltpu.VMEM((EDGES_PER_SUB,), jnp.int32),
        pltpu.VMEM((CHUNK,), jnp.int32),
        pltpu.VMEM((CHUNK,), jnp.int32),
        pltpu.VMEM((CHUNK,), jnp.int32),
        pltpu.VMEM((CHUNK, DH), jnp.float32),
        pltpu.VMEM((CHUNK, DH), jnp.float32),
        pltpu.VMEM((CHUNK, DH), jnp.float32),
        pltpu.VMEM_SHARED((N_NODES, DH), jnp.float32),
        pltpu.SemaphoreType.DMA,
        pltpu.SemaphoreType.DMA,
        pltpu.SemaphoreType.DMA,
        pltpu.SemaphoreType.DMA,
        pltpu.SemaphoreType.DMA,
        pltpu.SemaphoreType.DMA,
    ],
)
def _sc_aggregate(t0_hbm, t1_hbm, r0_hbm, r1_hbm, src_hbm, dst_hbm,
                  out0_hbm, out1_hbm, idx_src, d0, d1, d2, rb0, rb1, rb2,
                  shared, sg0, sg1, sg2, si0, si1, si2):
    c = lax.axis_index("c")
    s = lax.axis_index("s")
    row0 = s * ROWS_PER_SUB

    tail0 = NS * ROWS_PER_SUB

    @pl.when(c == 0)
    def _():
        pltpu.async_copy(r0_hbm.at[pl.ds(row0, ROWS_PER_SUB)],
                         shared.at[pl.ds(row0, ROWS_PER_SUB)], sg0)

        @pl.when(s == NS - 1)
        def _():
            pltpu.async_copy(r0_hbm.at[pl.ds(tail0, ROWS_TAIL)],
                             shared.at[pl.ds(tail0, ROWS_TAIL)], sg1)

    @pl.when(c == 1)
    def _():
        pltpu.async_copy(r1_hbm.at[pl.ds(row0, ROWS_PER_SUB)],
                         shared.at[pl.ds(row0, ROWS_PER_SUB)], sg0)

        @pl.when(s == NS - 1)
        def _():
            pltpu.async_copy(r1_hbm.at[pl.ds(tail0, ROWS_TAIL)],
                             shared.at[pl.ds(tail0, ROWS_TAIL)], sg1)

    pltpu.sync_copy(src_hbm.at[s], idx_src)
    pltpu.make_async_copy(r0_hbm.at[pl.ds(row0, ROWS_PER_SUB)],
                          shared.at[pl.ds(row0, ROWS_PER_SUB)], sg0).wait()

    @pl.when(s == NS - 1)
    def _():
        pltpu.make_async_copy(r0_hbm.at[pl.ds(tail0, ROWS_TAIL)],
                              shared.at[pl.ds(tail0, ROWS_TAIL)], sg1).wait()

    plsc.subcore_barrier()

    dbufs = (d0, d1, d2)
    rbufs = (rb0, rb1, rb2)
    gsems = (sg0, sg1, sg2)
    isems = (si0, si1, si2)

    def start_gather(ch, buf, sem):
        sl = idx_src.at[pl.ds(ch * CHUNK, CHUNK)]

        @pl.when(c == 0)
        def _():
            pltpu.async_copy(t0_hbm.at[sl], buf, sem)

        @pl.when(c == 1)
        def _():
            pltpu.async_copy(t1_hbm.at[sl], buf, sem)

    def wait_gather(ch, buf, sem):
        sl = idx_src.at[pl.ds(ch * CHUNK, CHUNK)]
        pltpu.make_async_copy(t0_hbm.at[sl], buf, sem).wait()

    def start_didx(ch, buf, sem):
        sl = dst_hbm.at[pl.ds(s * EDGES_PER_SUB + ch * CHUNK, CHUNK)]
        pltpu.async_copy(sl, buf, sem)

    def wait_didx(ch, buf, sem):
        sl = dst_hbm.at[pl.ds(s * EDGES_PER_SUB + ch * CHUNK, CHUNK)]
        pltpu.make_async_copy(sl, buf, sem).wait()

    for p in range(3):
        start_didx(p, dbufs[p], isems[p])
        start_gather(p, rbufs[p], gsems[p])

    def step(ch, b, prefetch):
        wait_gather(ch, rbufs[b], gsems[b])
        wait_didx(ch, dbufs[b], isems[b])
        pltpu.sync_copy(rbufs[b], shared.at[dbufs[b]], add=True)
        if prefetch:
            @pl.when(ch + 3 < N_CHUNKS)
            def _():
                start_didx(ch + 3, dbufs[b], isems[b])
                start_gather(ch + 3, rbufs[b], gsems[b])

    def body(g, carry):
        ch0 = g * 3
        step(ch0, 0, True)
        step(ch0 + 1, 1, True)
        step(ch0 + 2, 2, True)
        return carry

    lax.fori_loop(0, N_CHUNKS // 3, body, 0)

    step(N_CHUNKS - 2, (N_CHUNKS - 2) % 3, False)
    step(N_CHUNKS - 1, (N_CHUNKS - 1) % 3, False)

    plsc.subcore_barrier()

    @pl.when(c == 0)
    def _():
        pltpu.sync_copy(shared.at[pl.ds(row0, ROWS_PER_SUB)],
                        out0_hbm.at[pl.ds(row0, ROWS_PER_SUB)])

        @pl.when(s == NS - 1)
        def _():
            pltpu.sync_copy(shared.at[pl.ds(tail0, ROWS_TAIL)],
                            out0_hbm.at[pl.ds(tail0, ROWS_TAIL)])

    @pl.when(c == 1)
    def _():
        pltpu.sync_copy(shared.at[pl.ds(row0, ROWS_PER_SUB)],
                        out1_hbm.at[pl.ds(row0, ROWS_PER_SUB)])

        @pl.when(s == NS - 1)
        def _():
            pltpu.sync_copy(shared.at[pl.ds(tail0, ROWS_TAIL)],
                            out1_hbm.at[pl.ds(tail0, ROWS_TAIL)])


def _pool_mlp_body(a0_ref, a1_ref, batch_ref,
                   w1_ref, b1_ref, g1_ref, be1_ref,
                   w2_ref, b2_ref, g2_ref, be2_ref,
                   out_ref, pooled_acc, counts_acc):
    i = pl.program_id(0)

    @pl.when(i == 0)
    def _():
        pooled_acc[...] = jnp.zeros_like(pooled_acc)
        counts_acc[...] = jnp.zeros_like(counts_acc)

    h = jnp.maximum(
        jnp.concatenate([a0_ref[...], a1_ref[...]], axis=1), 0.0)
    b_ids = batch_ref[0, 0, :]
    gids = lax.broadcasted_iota(jnp.int32, (ROWS_BLK, N_GRAPHS), 1)
    m = (b_ids[:, None] == gids).astype(jnp.float32)
    pooled_acc[...] += lax.dot_general(
        m, h, (((0,), (0,)), ((), ())), precision=lax.Precision.HIGHEST,
        preferred_element_type=jnp.float32)
    counts_acc[...] += lax.dot_general(
        m, jnp.ones((ROWS_BLK, D), jnp.float32),
        (((0,), (0,)), ((), ())), precision=lax.Precision.HIGHEST,
        preferred_element_type=jnp.float32)

    @pl.when(i == N_BLKS - 1)
    def _():
        pooled = pooled_acc[...] / jnp.maximum(counts_acc[...], 1.0)
        h1 = jnp.maximum(
            jnp.dot(pooled, w1_ref[...], preferred_element_type=jnp.float32)
            + b1_ref[...], 0.0)
        mu1 = jnp.mean(h1, axis=0, keepdims=True)
        var1 = jnp.mean((h1 - mu1) ** 2, axis=0, keepdims=True)
        h1n = (h1 - mu1) / jnp.sqrt(var1 + EPS) * g1_ref[...] + be1_ref[...]
        h2 = jnp.maximum(
            jnp.dot(h1n, w2_ref[...], preferred_element_type=jnp.float32)
            + b2_ref[...], 0.0)
        mu2 = jnp.mean(h2, axis=0, keepdims=True)
        var2 = jnp.mean((h2 - mu2) ** 2, axis=0, keepdims=True)
        h2n = (h2 - mu2) / jnp.sqrt(var2 + EPS) * g2_ref[...] + be2_ref[...]
        mx = jnp.max(h2n, axis=1, keepdims=True)
        lse = jnp.log(jnp.sum(jnp.exp(h2n - mx), axis=1, keepdims=True))
        out_ref[...] = h2n - mx - lse


def _pool_mlp(a0, a1, batch3, w1, b1, g1, be1, w2, b2, g2, be2):
    vec_d = pl.BlockSpec((1, D), lambda i: (0, 0))
    vec_c = pl.BlockSpec((1, N_CLASSES), lambda i: (0, 0))
    return pl.pallas_call(
        _pool_mlp_body,
        grid=(N_BLKS,),
        in_specs=[
            _half_spec, _half_spec,
            pl.BlockSpec((1, 1, ROWS_BLK), lambda i: (i, 0, 0)),
            pl.BlockSpec((D, D), lambda i: (0, 0)), vec_d, vec_d, vec_d,
            pl.BlockSpec((D, N_CLASSES), lambda i: (0, 0)), vec_c, vec_c, vec_c,
        ],
        out_specs=pl.BlockSpec((N_GRAPHS, N_CLASSES), lambda i: (0, 0)),
        out_shape=jax.ShapeDtypeStruct((N_GRAPHS, N_CLASSES), jnp.float32),
        scratch_shapes=[pltpu.VMEM((N_GRAPHS, D), jnp.float32),
                        pltpu.VMEM((N_GRAPHS, D), jnp.float32)],
    )(a0, a1, batch3, w1, b1, g1, be1, w2, b2, g2, be2)


def kernel(x, edge_index, batch, enc_W_root, enc_W_rel, enc_b,
           mlp_W1, mlp_b1, mlp_g1, mlp_be1,
           mlp_W2, mlp_b2, mlp_g2, mlp_be2):
    src = edge_index[0].reshape(NS, EDGES_PER_SUB)
    dst = edge_index[1]
    batch3 = batch.reshape(N_BLKS, 1, ROWS_BLK)

    h0 = h1 = None
    for i in range(DEPTH):
        wrel = enc_W_rel[i]
        wroot = enc_W_root[i]
        b = enc_b[i].reshape(1, D)
        if i == 0:
            t0, t1, r0, r1 = _enc_matmul_single(x, wrel, wroot, b)
        else:
            t0, t1, r0, r1 = _enc_matmul_halves(h0, h1, wrel, wroot, b)
        h0, h1 = _sc_aggregate(t0, t1, r0, r1, src, dst)

    return _pool_mlp(
        h0, h1, batch3,
        mlp_W1, mlp_b1.reshape(1, D),
        mlp_g1.reshape(1, D), mlp_be1.reshape(1, D),
        mlp_W2, mlp_b2.reshape(1, N_CLASSES), mlp_g2.reshape(1, N_CLASSES),
        mlp_be2.reshape(1, N_CLASSES))
